# baseline (speedup 1.0000x reference)
import functools

import jax
import jax.numpy as jnp
from jax import lax
from jax.experimental import pallas as pl
from jax.experimental.pallas import tpu as pltpu
from jax.experimental.pallas import tpu_sc as plsc

S = 2048
CLIP = 64
D = 64
T = 2 * CLIP + 1
RC = 512
CL = 2 * RC + T
NCHUNK = S // RC
NW = 32
ROWS_PER_W = S // NW


def _build_sc_kernel():
    mesh = plsc.VectorSubcoreMesh(core_axis_name="c", subcore_axis_name="s")

    @functools.partial(
        pl.kernel,
        mesh=mesh,
        out_type=jax.ShapeDtypeStruct((S, S * D), jnp.float32),
        scratch_types=[
            pltpu.VMEM((CL * D,), jnp.float32),
            pltpu.SemaphoreType.DMA,
        ],
        compiler_params=pltpu.CompilerParams(use_tc_tiling_on_sc=False),
    )
    def sc_kernel(table_hbm, out_hbm, cbuf, sem):
        cid = lax.axis_index("c")
        sid = lax.axis_index("s")
        wid = sid * 2 + cid

        pltpu.sync_copy(table_hbm, cbuf.at[pl.ds(RC * D, T * D)])

        def fill_body(k, _):
            for l in range(D // 16):
                v0 = cbuf[pl.ds(RC * D + l * 16, 16)]
                cbuf[pl.ds(k * D + l * 16, 16)] = v0
                v1 = cbuf[pl.ds((RC + T - 1) * D + l * 16, 16)]
                cbuf[pl.ds((RC + T) * D + k * D + l * 16, 16)] = v1
            return 0

        lax.fori_loop(0, RC, fill_body, 0)

        i0 = wid * ROWS_PER_W

        def row_body(r, _):
            i = i0 + r
            for c in range(NCHUNK):
                s0 = S - i + c * RC
                src = jnp.clip(s0 - (S - CLIP - RC), 0, RC + T)
                pltpu.async_copy(
                    cbuf.at[pl.ds(src * D, RC * D)],
                    out_hbm.at[i, pl.ds(c * RC * D, RC * D)],
                    sem,
                )
            return 0

        lax.fori_loop(0, ROWS_PER_W, row_body, 0)

        def drain_body(r, _):
            for c in range(NCHUNK):
                pltpu.make_async_copy(
                    cbuf.at[pl.ds(0, RC * D)],
                    out_hbm.at[i0, pl.ds(c * RC * D, RC * D)],
                    sem,
                ).wait()
            return 0

        lax.fori_loop(0, ROWS_PER_W, drain_body, 0)

    return sc_kernel


def kernel(x, encoding_matrix):
    del x
    out = _build_sc_kernel()(encoding_matrix.reshape(T * D))
    return out.reshape(S, S, D)


# speedup vs baseline: 5.8174x; 5.8174x over previous
import functools

import jax
import jax.numpy as jnp
from jax import lax
from jax.experimental import pallas as pl
from jax.experimental.pallas import tpu as pltpu
from jax.experimental.pallas import tpu_sc as plsc

S = 2048
CLIP = 64
D = 64
T = 2 * CLIP + 1
NTD = D // 8
NTJ = S // 128
KW = 400
SRC_MAX = 272
BAND_LO = 128
BAND_HI = 272
NW = 32
ROWS_PER_W = S // NW
NPH = 8
RPG = ROWS_PER_W // NPH


def _build_sc_kernel():
    mesh = plsc.VectorSubcoreMesh(core_axis_name="c", subcore_axis_name="s")

    @functools.partial(
        pl.kernel,
        mesh=mesh,
        out_type=jax.ShapeDtypeStruct((S, NTD, NTJ, 8, 128), jnp.float32),
        scratch_types=[
            pltpu.VMEM((T * D,), jnp.float32),
            pltpu.VMEM((NTD, 8, KW), jnp.float32),
            pltpu.VMEM((NTD, 8, KW), jnp.float32),
            pltpu.SemaphoreType.DMA,
        ],
        compiler_params=pltpu.CompilerParams(
            use_tc_tiling_on_sc=False, needs_layout_passes=False
        ),
    )
    def sc_kernel(table_hbm, out_hbm, tbl, cimg_a, cimg_b, sem):
        cid = lax.axis_index("c")
        sid = lax.axis_index("s")
        wid = sid * 2 + cid
        bufs = (cimg_a, cimg_b)

        pltpu.sync_copy(table_hbm, tbl)

        lane = lax.iota(jnp.int32, 16)

        def build(buf, p, m_lo, m_hi):
            ng = (m_hi - m_lo) // 16

            def body(g, _):
                dcomb = g // ng
                mg = g % ng
                m = m_lo + mg * 16 + lane
                row = jnp.clip(m + p - 136, 0, T - 1)
                vals = plsc.load_gather(tbl, [row * D + dcomb])
                buf[dcomb // 8, dcomb % 8, pl.ds(m_lo + mg * 16, 16)] = vals
                return 0

            lax.fori_loop(0, D * ng, body, 0)

        def drain_group():
            def body(q, _):
                pltpu.make_async_copy(
                    bufs[0].at[:, :, pl.ds(0, 128)],
                    out_hbm.at[i0, :, 0],
                    sem,
                ).wait()
                return 0

            lax.fori_loop(0, RPG * NTJ, body, 0)

        i0 = wid * ROWS_PER_W

        build(bufs[0], 0, 0, KW)
        build(bufs[1], 7, 0, KW)

        for g in range(NPH):
            p = (-g) % NPH
            buf = bufs[g % 2]

            def issue_body(q, _, g=g, p=p, buf=buf):
                rr = q // NTJ
                tj = q % NTJ
                i = i0 + rr * NPH + g
                src = jnp.clip(S - i + tj * 128 - 1848 - p, 0, SRC_MAX)
                src = pl.multiple_of(src, 8)
                pltpu.async_copy(
                    buf.at[:, :, pl.ds(src, 128)],
                    out_hbm.at[i, :, tj],
                    sem,
                )
                return 0

            lax.fori_loop(0, RPG * NTJ, issue_body, 0)

            if g >= 1:
                drain_group()
            if g < NPH - 1:
                build(bufs[(g + 1) % 2], (-(g + 1)) % NPH, BAND_LO, BAND_HI)

        drain_group()

    return sc_kernel


def kernel(x, encoding_matrix):
    del x
    b = _build_sc_kernel()(encoding_matrix.reshape(T * D))
    return b.transpose(0, 2, 4, 1, 3).reshape(S, S, D)
